# Initial kernel scaffold; baseline (speedup 1.0000x reference)
import jax
import jax.numpy as jnp
from jax.experimental import pallas as pl


def kernel(x, edge_index, edge_type, params):
    raise NotImplementedError("write your pallas kernel here")


# speedup vs baseline: 3.8188x; 3.8188x over previous
import functools

import jax
import jax.numpy as jnp
from jax import lax
from jax.experimental import pallas as pl
from jax.experimental.pallas import tpu as pltpu
from jax.experimental.pallas import tpu_sc as plsc

N = 10000
E = 160000
NODE_DIM = 128
HID = 256
R = 4
NB = 8
L = 3

NSEG = N * R
NC = 2
NS = 16
G = 6
SG = 6784
EPT = E // NS
CH = 128
CAP = EPT + 2 * CH
FW = 128
PADROW = N


@functools.lru_cache(maxsize=None)
def _make_sc_segment_sum(nparts, with_cnt):
    mesh = plsc.VectorSubcoreMesh(
        core_axis_name="c", subcore_axis_name="s", num_cores=NC, num_subcores=NS
    )
    out_type = [jax.ShapeDtypeStruct((G * SG, nparts * FW), jnp.float32)]
    if with_cnt:
        out_type.append(jax.ShapeDtypeStruct((G * SG,), jnp.float32))

    scratch = dict(
        src_v=pltpu.VMEM((EPT,), jnp.int32),
        seg_full=pltpu.VMEM((EPT,), jnp.int32),
        idx_v=pltpu.VMEM((CAP,), jnp.int32),
        seg_v=pltpu.VMEM((CAP,), jnp.int32),
        idx_ch=pltpu.VMEM((CH,), jnp.int32),
        seg_ch=pltpu.VMEM((CH,), jnp.int32),
        rows_v=pltpu.VMEM((CH, FW), jnp.float32),
        zrows_v=pltpu.VMEM((32, FW), jnp.float32),
        acc_sh=pltpu.VMEM_SHARED((SG + 8, FW), jnp.float32),
        sem=pltpu.SemaphoreType.DMA,
    )
    if with_cnt:
        scratch.update(
            ones_v=pltpu.VMEM((CH,), jnp.float32),
            zcol_v=pltpu.VMEM((432,), jnp.float32),
            cnt_v=pltpu.VMEM((432,), jnp.float32),
            cnt_sh=pltpu.VMEM_SHARED((SG + 8,), jnp.float32),
        )

    def body(src_hbm, dst_hbm, typ_hbm, *rest):
        h_parts = rest[:nparts]
        rest = rest[nparts:]
        s_out = rest[0]
        if with_cnt:
            cnt_out = rest[1]
            scr = dict(zip(scratch.keys(), rest[2:]))
        else:
            scr = dict(zip(scratch.keys(), rest[1:]))
        src_v, seg_full = scr["src_v"], scr["seg_full"]
        idx_v, seg_v = scr["idx_v"], scr["seg_v"]
        idx_ch, seg_ch = scr["idx_ch"], scr["seg_ch"]
        rows_v, zrows_v, acc_sh, sem = (
            scr["rows_v"], scr["zrows_v"], scr["acc_sh"], scr["sem"]
        )

        c = lax.axis_index("c")
        s = lax.axis_index("s")
        ebase = s * EPT
        pltpu.sync_copy(src_hbm.at[pl.ds(ebase, EPT)], src_v)
        pltpu.sync_copy(dst_hbm.at[pl.ds(ebase, EPT)], seg_full)
        pltpu.sync_copy(typ_hbm.at[pl.ds(ebase, EPT)], idx_v.at[pl.ds(0, EPT)])

        z16 = jnp.zeros((16,), jnp.float32)
        one16 = jnp.ones((16,), jnp.float32)
        zpad = jnp.zeros((16,), jnp.int32)
        tpad = jnp.full((16,), SG, jnp.int32)
        iota16 = lax.iota(jnp.int32, 16)
        all_true = jnp.full((16,), True)

        def segfill(i, _):
            sl = pl.ds(i * 16, 16)
            seg_full[sl] = seg_full[sl] * R + idx_v[sl]
            return 0

        lax.fori_loop(0, EPT // 16, segfill, 0)

        def zfill(i, _):
            for j in range(FW // 16):
                zrows_v[i, pl.ds(j * 16, 16)] = z16
            return 0

        lax.fori_loop(0, 32, zfill, 0)
        if with_cnt:
            for j in range(CH // 16):
                scr["ones_v"][pl.ds(j * 16, 16)] = one16

            def cfill(i, _):
                scr["zcol_v"][pl.ds(i * 16, 16)] = z16
                return 0

            lax.fori_loop(0, 27, cfill, 0)

        wpt = SG // NS
        for gi in range(G // NC):
            g = c * (G // NC) + gi
            lo = g * SG

            def scan_body(i, n):
                sl = pl.ds(i * 16, 16)
                seg = seg_full[sl]
                s16 = src_v[sl]
                m = (seg >= lo) & (seg < lo + SG)
                mi = jnp.where(m, 1, 0)
                pos = n + plsc.cumsum(mi) - 1
                plsc.store_scatter(idx_v, [pos], s16, mask=m)
                plsc.store_scatter(seg_v, [pos], seg - lo, mask=m)
                return n + jnp.sum(mi)

            n = lax.fori_loop(0, EPT // 16, scan_body, jnp.int32(0))

            for j in range(CH // 16):
                plsc.store_scatter(idx_v, [n + j * 16 + iota16], zpad,
                                   mask=all_true)
                plsc.store_scatter(seg_v, [n + j * 16 + iota16], tpad,
                                   mask=all_true)
            nch = (n + CH - 1) // CH

            for f in range(nparts):
                zb = s * wpt
                for z in range(13):
                    pltpu.sync_copy(zrows_v, acc_sh.at[pl.ds(zb + z * 32, 32)])
                pltpu.sync_copy(zrows_v.at[pl.ds(0, 8)],
                                acc_sh.at[pl.ds(zb + 416, 8)])
                if with_cnt:
                    pltpu.sync_copy(scr["zcol_v"].at[pl.ds(0, wpt)],
                                    scr["cnt_sh"].at[pl.ds(zb, wpt)])
                plsc.subcore_barrier()

                def flush_body(ci, _):
                    for j in range(CH // 16):
                        sl = pl.ds(j * 16, 16)
                        idx_ch[sl] = idx_v[pl.ds(ci * CH + j * 16, 16)]
                        seg_ch[sl] = seg_v[pl.ds(ci * CH + j * 16, 16)]
                    pltpu.async_copy(h_parts[f].at[idx_ch], rows_v, sem).wait()
                    pltpu.sync_copy(rows_v, acc_sh.at[seg_ch], add=True)
                    if with_cnt:
                        pltpu.sync_copy(scr["ones_v"],
                                        scr["cnt_sh"].at[seg_ch], add=True)
                    return 0

                lax.fori_loop(0, nch, flush_body, 0)
                plsc.subcore_barrier()

                wb = s * wpt
                pltpu.sync_copy(
                    acc_sh.at[pl.ds(wb, wpt)],
                    s_out.at[pl.ds(g * SG + wb, wpt), pl.ds(f * FW, FW)])
                if with_cnt:
                    pltpu.sync_copy(scr["cnt_sh"].at[pl.ds(wb, wpt)],
                                    scr["cnt_v"].at[pl.ds(0, wpt)])
                    pltpu.sync_copy(scr["cnt_v"].at[pl.ds(0, wpt)],
                                    cnt_out.at[pl.ds(g * SG + wb, wpt)])
                plsc.subcore_barrier()

    return pl.kernel(
        body, out_type=out_type, mesh=mesh, scratch_types=list(scratch.values()),
        compiler_params=pltpu.CompilerParams(needs_layout_passes=False),
    )


@functools.lru_cache(maxsize=None)
def _make_wcat(in_dim):
    def body(comb_ref, bases_ref, out_ref):
        out_ref[...] = jnp.dot(comb_ref[...], bases_ref[...],
                               preferred_element_type=jnp.float32)

    return pl.pallas_call(
        body,
        out_shape=jax.ShapeDtypeStruct((R, in_dim * HID), jnp.float32),
        grid=(8,),
        in_specs=[
            pl.BlockSpec((R, NB), lambda i: (0, 0)),
            pl.BlockSpec((NB, in_dim * HID // 8), lambda i: (0, i)),
        ],
        out_specs=pl.BlockSpec((R, in_dim * HID // 8), lambda i: (0, i)),
    )


BN = 400


@functools.lru_cache(maxsize=None)
def _make_layer(in_dim, residual):
    def body(s_ref, cnt_ref, h_ref, wcat_ref, root_ref, bias_ref, lnw_ref,
             lnb_ref, out_ref):
        inv = 1.0 / jnp.maximum(cnt_ref[...], 1.0)
        acc = jnp.dot(h_ref[...], root_ref[...],
                      preferred_element_type=jnp.float32)
        for r in range(R):
            sc = s_ref[:, r * in_dim:(r + 1) * in_dim] * inv[:, r:r + 1]
            acc += jnp.dot(sc, wcat_ref[r * in_dim:(r + 1) * in_dim, :],
                           preferred_element_type=jnp.float32)
        hnew = acc + bias_ref[...]
        mu = jnp.mean(hnew, axis=-1, keepdims=True)
        var = jnp.mean((hnew - mu) ** 2, axis=-1, keepdims=True)
        hnew = (hnew - mu) / jnp.sqrt(var + 1e-5) * lnw_ref[...] + lnb_ref[...]
        hnew = jnp.where(hnew > 0, hnew, jnp.exp(jnp.minimum(hnew, 0.0)) - 1.0)
        if residual:
            hnew = hnew + h_ref[...]
        out_ref[...] = hnew

    return pl.pallas_call(
        body,
        out_shape=jax.ShapeDtypeStruct((N, HID), jnp.float32),
        grid=(N // BN,),
        in_specs=[
            pl.BlockSpec((BN, R * in_dim), lambda i: (i, 0)),
            pl.BlockSpec((BN, R), lambda i: (i, 0)),
            pl.BlockSpec((BN, in_dim), lambda i: (i, 0)),
            pl.BlockSpec((R * in_dim, HID), lambda i: (0, 0)),
            pl.BlockSpec((in_dim, HID), lambda i: (0, 0)),
            pl.BlockSpec((1, HID), lambda i: (0, 0)),
            pl.BlockSpec((1, HID), lambda i: (0, 0)),
            pl.BlockSpec((1, HID), lambda i: (0, 0)),
        ],
        out_specs=pl.BlockSpec((BN, HID), lambda i: (i, 0)),
    )


def _head_body(h_ref, ee_ref, w1_ref, b1_ref, w2_ref, b2_ref, out_ref, acc):
    i = pl.program_id(0)

    @pl.when(i == 0)
    def _():
        acc[...] = jnp.zeros_like(acc)

    acc[...] += jnp.sum(h_ref[...], axis=0, keepdims=True)

    @pl.when(i == N // BN - 1)
    def _():
        rep = acc[...] / N
        rep4 = jnp.broadcast_to(rep, (R, HID))
        att_in = jnp.concatenate([rep4, ee_ref[...]], axis=1)
        hid = jnp.maximum(
            jnp.dot(att_in, w1_ref[...], preferred_element_type=jnp.float32)
            + b1_ref[...], 0.0)
        logits = jnp.dot(hid, w2_ref[...],
                         preferred_element_type=jnp.float32) + b2_ref[...]
        mx = jnp.max(logits, axis=-1, keepdims=True)
        ex = jnp.exp(logits - mx)
        out_ref[...] = ex / jnp.sum(ex, axis=-1, keepdims=True)


_head_call = pl.pallas_call(
    _head_body,
    out_shape=jax.ShapeDtypeStruct((R, R), jnp.float32),
    grid=(N // BN,),
    in_specs=[
        pl.BlockSpec((BN, HID), lambda i: (i, 0)),
        pl.BlockSpec((R, HID), lambda i: (0, 0)),
        pl.BlockSpec((2 * HID, HID), lambda i: (0, 0)),
        pl.BlockSpec((1, HID), lambda i: (0, 0)),
        pl.BlockSpec((HID, R), lambda i: (0, 0)),
        pl.BlockSpec((1, R), lambda i: (0, 0)),
    ],
    out_specs=pl.BlockSpec((R, R), lambda i: (0, 0)),
    scratch_shapes=[pltpu.VMEM((1, HID), jnp.float32)],
)


def kernel(x, edge_index, edge_type, params):
    src = edge_index[0]
    dst = edge_index[1]

    h = x
    cnt2d = None
    for i in range(L):
        in_dim = NODE_DIM if i == 0 else HID
        nparts = in_dim // FW
        parts = [h[:, f * FW:(f + 1) * FW] for f in range(nparts)]
        sck = _make_sc_segment_sum(nparts, i == 0)
        if i == 0:
            s_raw, cnt_raw = sck(src, dst, edge_type, *parts)
            cnt2d = cnt_raw[:NSEG].reshape(N, R)
        else:
            (s_raw,) = sck(src, dst, edge_type, *parts)
        s2d = s_raw[:NSEG].reshape(N, R * in_dim)

        bases2d = params['bases_%d' % i].reshape(NB, in_dim * HID)
        wcat = _make_wcat(in_dim)(params['comb_%d' % i], bases2d)
        wcat = wcat.reshape(R * in_dim, HID)

        h = _make_layer(in_dim, i > 0)(
            s2d, cnt2d, h, wcat, params['root_%d' % i],
            params['bias_%d' % i].reshape(1, HID),
            params['ln_w_%d' % i].reshape(1, HID),
            params['ln_b_%d' % i].reshape(1, HID),
        )

    scores = _head_call(
        h, params['edge_embed'], params['att_w1'],
        params['att_b1'].reshape(1, HID), params['att_w2'],
        params['att_b2'].reshape(1, R),
    )
    return h, scores


# speedup vs baseline: 4.0229x; 1.0534x over previous
import functools

import jax
import jax.numpy as jnp
from jax import lax
from jax.experimental import pallas as pl
from jax.experimental.pallas import tpu as pltpu
from jax.experimental.pallas import tpu_sc as plsc

N = 10000
E = 160000
NODE_DIM = 128
HID = 256
R = 4
NB = 8
L = 3

NSEG = N * R
NC = 2
NS = 16
G = 6
SG = 6784
EPT = E // NS
CH = 128
CAP = EPT + 2 * CH
FW = 128
PADROW = N


@functools.lru_cache(maxsize=None)
def _make_sc_segment_sum(nparts, with_cnt):
    nbuf = 1 if with_cnt else 2
    mesh = plsc.VectorSubcoreMesh(
        core_axis_name="c", subcore_axis_name="s", num_cores=NC, num_subcores=NS
    )
    out_type = [jax.ShapeDtypeStruct((G * SG, nparts * FW), jnp.float32)]
    if with_cnt:
        out_type.append(jax.ShapeDtypeStruct((G * SG,), jnp.float32))

    scratch = dict(
        src_v=pltpu.VMEM((EPT,), jnp.int32),
        seg_full=pltpu.VMEM((EPT,), jnp.int32),
        idx_v=pltpu.VMEM((CAP,), jnp.int32),
        seg2d=pltpu.VMEM((CAP // CH, CH), jnp.int32),
        rows_v=pltpu.VMEM((nbuf, CH, FW), jnp.float32),
        zrows_v=pltpu.VMEM((16, FW), jnp.float32),
        acc_sh=pltpu.VMEM_SHARED((SG + 8, FW), jnp.float32),
        sem0=pltpu.SemaphoreType.DMA,
        sem1=pltpu.SemaphoreType.DMA,
    )
    if with_cnt:
        scratch.update(
            ones_v=pltpu.VMEM((CH,), jnp.float32),
            cnt_v=pltpu.VMEM((432,), jnp.float32),
            cnt_sh=pltpu.VMEM_SHARED((SG + 8,), jnp.float32),
        )

    def body(src_hbm, dst_hbm, typ_hbm, *rest):
        h_parts = rest[:nparts]
        rest = rest[nparts:]
        s_out = rest[0]
        if with_cnt:
            cnt_out = rest[1]
            scr = dict(zip(scratch.keys(), rest[2:]))
        else:
            scr = dict(zip(scratch.keys(), rest[1:]))
        src_v, seg_full = scr["src_v"], scr["seg_full"]
        idx_v, seg2d = scr["idx_v"], scr["seg2d"]
        rows_v, zrows_v, acc_sh = scr["rows_v"], scr["zrows_v"], scr["acc_sh"]
        sems = [scr["sem0"], scr["sem1"]]

        c = lax.axis_index("c")
        s = lax.axis_index("s")
        ebase = s * EPT
        pltpu.sync_copy(src_hbm.at[pl.ds(ebase, EPT)], src_v)
        pltpu.sync_copy(dst_hbm.at[pl.ds(ebase, EPT)], seg_full)
        pltpu.sync_copy(typ_hbm.at[pl.ds(ebase, EPT)], idx_v.at[pl.ds(0, EPT)])

        z16 = jnp.zeros((16,), jnp.float32)
        one16 = jnp.ones((16,), jnp.float32)
        zpad = jnp.zeros((16,), jnp.int32)
        tpad = jnp.full((16,), SG, jnp.int32)
        iota16 = lax.iota(jnp.int32, 16)
        all_true = jnp.full((16,), True)

        def segfill(i, _):
            sl = pl.ds(i * 16, 16)
            seg_full[sl] = seg_full[sl] * R + idx_v[sl]
            return 0

        lax.fori_loop(0, EPT // 16, segfill, 0)

        def zfill(i, _):
            for j in range(FW // 16):
                zrows_v[i, pl.ds(j * 16, 16)] = z16
            return 0

        lax.fori_loop(0, 16, zfill, 0)
        if with_cnt:
            for j in range(CH // 16):
                scr["ones_v"][pl.ds(j * 16, 16)] = one16

            def cfill(i, _):
                scr["cnt_v"][pl.ds(i * 16, 16)] = z16
                return 0

            lax.fori_loop(0, 27, cfill, 0)

        wpt = SG // NS
        for gi in range(G // NC):
            g = c * (G // NC) + gi
            lo = g * SG

            def scan_body(i, n):
                sl = pl.ds(i * 16, 16)
                seg = seg_full[sl]
                s16 = src_v[sl]
                m = (seg >= lo) & (seg < lo + SG)
                mi = jnp.where(m, 1, 0)
                pos = n + plsc.cumsum(mi) - 1
                plsc.store_scatter(idx_v, [pos], s16, mask=m)
                plsc.store_scatter(
                    seg2d,
                    [lax.shift_right_logical(pos, 7), pos & (CH - 1)],
                    seg - lo, mask=m)
                return n + jnp.sum(mi)

            n = lax.fori_loop(0, EPT // 16, scan_body, jnp.int32(0))

            for j in range(CH // 16):
                p = n + j * 16 + iota16
                plsc.store_scatter(idx_v, [p], zpad, mask=all_true)
                plsc.store_scatter(
                    seg2d, [lax.shift_right_logical(p, 7), p & (CH - 1)],
                    tpad, mask=all_true)
            nch = (n + CH - 1) // CH

            for f in range(nparts):
                zb = s * wpt
                for z in range(26):
                    pltpu.sync_copy(zrows_v, acc_sh.at[pl.ds(zb + z * 16, 16)])
                pltpu.sync_copy(zrows_v.at[pl.ds(0, 8)],
                                acc_sh.at[pl.ds(zb + 416, 8)])
                if with_cnt:
                    pltpu.sync_copy(scr["cnt_v"].at[pl.ds(0, wpt)],
                                    scr["cnt_sh"].at[pl.ds(zb, wpt)])
                plsc.subcore_barrier()

                def chunk_idx(ci):
                    return idx_v.at[pl.ds(pl.multiple_of(ci * CH, CH), CH)]

                if nbuf == 1:
                    def flush_body(ci, _):
                        pltpu.async_copy(h_parts[f].at[chunk_idx(ci)],
                                         rows_v.at[0], sems[0]).wait()
                        pltpu.sync_copy(rows_v.at[0], acc_sh.at[seg2d.at[ci]],
                                        add=True)
                        if with_cnt:
                            pltpu.sync_copy(scr["ones_v"],
                                            scr["cnt_sh"].at[seg2d.at[ci]],
                                            add=True)
                        return 0

                    lax.fori_loop(0, nch, flush_body, 0)
                else:
                    @pl.when(nch > 0)
                    def _():
                        pltpu.async_copy(h_parts[f].at[chunk_idx(0)],
                                         rows_v.at[0], sems[0])

                    def pair_body(q, _):
                        for b in range(2):
                            ci = 2 * q + b
                            nb = 1 - b

                            @pl.when(ci < nch)
                            def _():
                                @pl.when(ci + 1 < nch)
                                def _():
                                    pltpu.async_copy(
                                        h_parts[f].at[chunk_idx(ci + 1)],
                                        rows_v.at[nb], sems[nb])

                                pltpu.make_async_copy(
                                    h_parts[f].at[chunk_idx(ci)],
                                    rows_v.at[b], sems[b]).wait()
                                pltpu.sync_copy(rows_v.at[b],
                                                acc_sh.at[seg2d.at[ci]],
                                                add=True)
                        return 0

                    lax.fori_loop(0, (nch + 1) // 2, pair_body, 0)
                plsc.subcore_barrier()

                wb = s * wpt
                pltpu.sync_copy(
                    acc_sh.at[pl.ds(wb, wpt)],
                    s_out.at[pl.ds(g * SG + wb, wpt), pl.ds(f * FW, FW)])
                if with_cnt:
                    pltpu.sync_copy(scr["cnt_sh"].at[pl.ds(wb, wpt)],
                                    scr["cnt_v"].at[pl.ds(0, wpt)])
                    pltpu.sync_copy(scr["cnt_v"].at[pl.ds(0, wpt)],
                                    cnt_out.at[pl.ds(g * SG + wb, wpt)])
                plsc.subcore_barrier()
                if with_cnt:
                    def refill(i, _):
                        scr["cnt_v"][pl.ds(i * 16, 16)] = z16
                        return 0

                    lax.fori_loop(0, 27, refill, 0)

    return pl.kernel(
        body, out_type=out_type, mesh=mesh, scratch_types=list(scratch.values()),
        compiler_params=pltpu.CompilerParams(needs_layout_passes=False),
    )


@functools.lru_cache(maxsize=None)
def _make_wcat(in_dim):
    def body(comb_ref, bases_ref, out_ref):
        out_ref[...] = jnp.dot(comb_ref[...], bases_ref[...],
                               preferred_element_type=jnp.float32)

    return pl.pallas_call(
        body,
        out_shape=jax.ShapeDtypeStruct((R, in_dim * HID), jnp.float32),
        grid=(8,),
        in_specs=[
            pl.BlockSpec((R, NB), lambda i: (0, 0)),
            pl.BlockSpec((NB, in_dim * HID // 8), lambda i: (0, i)),
        ],
        out_specs=pl.BlockSpec((R, in_dim * HID // 8), lambda i: (0, i)),
    )


BN = 400


@functools.lru_cache(maxsize=None)
def _make_layer(in_dim, residual):
    def body(s_ref, cnt_ref, h_ref, wcat_ref, root_ref, bias_ref, lnw_ref,
             lnb_ref, out_ref):
        inv = 1.0 / jnp.maximum(cnt_ref[...], 1.0)
        acc = jnp.dot(h_ref[...], root_ref[...],
                      preferred_element_type=jnp.float32)
        for r in range(R):
            sc = s_ref[:, r * in_dim:(r + 1) * in_dim] * inv[:, r:r + 1]
            acc += jnp.dot(sc, wcat_ref[r * in_dim:(r + 1) * in_dim, :],
                           preferred_element_type=jnp.float32)
        hnew = acc + bias_ref[...]
        mu = jnp.mean(hnew, axis=-1, keepdims=True)
        var = jnp.mean((hnew - mu) ** 2, axis=-1, keepdims=True)
        hnew = (hnew - mu) / jnp.sqrt(var + 1e-5) * lnw_ref[...] + lnb_ref[...]
        hnew = jnp.where(hnew > 0, hnew, jnp.exp(jnp.minimum(hnew, 0.0)) - 1.0)
        if residual:
            hnew = hnew + h_ref[...]
        out_ref[...] = hnew

    return pl.pallas_call(
        body,
        out_shape=jax.ShapeDtypeStruct((N, HID), jnp.float32),
        grid=(N // BN,),
        in_specs=[
            pl.BlockSpec((BN, R * in_dim), lambda i: (i, 0)),
            pl.BlockSpec((BN, R), lambda i: (i, 0)),
            pl.BlockSpec((BN, in_dim), lambda i: (i, 0)),
            pl.BlockSpec((R * in_dim, HID), lambda i: (0, 0)),
            pl.BlockSpec((in_dim, HID), lambda i: (0, 0)),
            pl.BlockSpec((1, HID), lambda i: (0, 0)),
            pl.BlockSpec((1, HID), lambda i: (0, 0)),
            pl.BlockSpec((1, HID), lambda i: (0, 0)),
        ],
        out_specs=pl.BlockSpec((BN, HID), lambda i: (i, 0)),
    )


def _head_body(h_ref, ee_ref, w1_ref, b1_ref, w2_ref, b2_ref, out_ref, acc):
    i = pl.program_id(0)

    @pl.when(i == 0)
    def _():
        acc[...] = jnp.zeros_like(acc)

    acc[...] += jnp.sum(h_ref[...], axis=0, keepdims=True)

    @pl.when(i == N // BN - 1)
    def _():
        rep = acc[...] / N
        rep4 = jnp.broadcast_to(rep, (R, HID))
        att_in = jnp.concatenate([rep4, ee_ref[...]], axis=1)
        hid = jnp.maximum(
            jnp.dot(att_in, w1_ref[...], preferred_element_type=jnp.float32)
            + b1_ref[...], 0.0)
        logits = jnp.dot(hid, w2_ref[...],
                         preferred_element_type=jnp.float32) + b2_ref[...]
        mx = jnp.max(logits, axis=-1, keepdims=True)
        ex = jnp.exp(logits - mx)
        out_ref[...] = ex / jnp.sum(ex, axis=-1, keepdims=True)


_head_call = pl.pallas_call(
    _head_body,
    out_shape=jax.ShapeDtypeStruct((R, R), jnp.float32),
    grid=(N // BN,),
    in_specs=[
        pl.BlockSpec((BN, HID), lambda i: (i, 0)),
        pl.BlockSpec((R, HID), lambda i: (0, 0)),
        pl.BlockSpec((2 * HID, HID), lambda i: (0, 0)),
        pl.BlockSpec((1, HID), lambda i: (0, 0)),
        pl.BlockSpec((HID, R), lambda i: (0, 0)),
        pl.BlockSpec((1, R), lambda i: (0, 0)),
    ],
    out_specs=pl.BlockSpec((R, R), lambda i: (0, 0)),
    scratch_shapes=[pltpu.VMEM((1, HID), jnp.float32)],
)


def kernel(x, edge_index, edge_type, params):
    src = edge_index[0]
    dst = edge_index[1]

    h = x
    cnt2d = None
    for i in range(L):
        in_dim = NODE_DIM if i == 0 else HID
        nparts = in_dim // FW
        parts = [h[:, f * FW:(f + 1) * FW] for f in range(nparts)]
        sck = _make_sc_segment_sum(nparts, i == 0)
        if i == 0:
            s_raw, cnt_raw = sck(src, dst, edge_type, *parts)
            cnt2d = cnt_raw[:NSEG].reshape(N, R)
        else:
            (s_raw,) = sck(src, dst, edge_type, *parts)
        s2d = s_raw[:NSEG].reshape(N, R * in_dim)

        bases2d = params['bases_%d' % i].reshape(NB, in_dim * HID)
        wcat = _make_wcat(in_dim)(params['comb_%d' % i], bases2d)
        wcat = wcat.reshape(R * in_dim, HID)

        h = _make_layer(in_dim, i > 0)(
            s2d, cnt2d, h, wcat, params['root_%d' % i],
            params['bias_%d' % i].reshape(1, HID),
            params['ln_w_%d' % i].reshape(1, HID),
            params['ln_b_%d' % i].reshape(1, HID),
        )

    scores = _head_call(
        h, params['edge_embed'], params['att_w1'],
        params['att_b1'].reshape(1, HID), params['att_w2'],
        params['att_b2'].reshape(1, R),
    )
    return h, scores
